# baseline (speedup 1.0000x reference)
import jax
import jax.numpy as jnp
from jax.experimental import pallas as pl


def _add_kernel(x_ref, w_ref, o_ref):
    o_ref[...] = x_ref[...] + w_ref[...][None, :, :]


def kernel(x, encoding_weight):
    batch, seq_len, d_model = x.shape
    sblk = 256
    grid = (seq_len // sblk,)
    return pl.pallas_call(
        _add_kernel,
        grid=grid,
        in_specs=[
            pl.BlockSpec((batch, sblk, d_model), lambda i: (0, i, 0)),
            pl.BlockSpec((sblk, d_model), lambda i: (i, 0)),
        ],
        out_specs=pl.BlockSpec((batch, sblk, d_model), lambda i: (0, i, 0)),
        out_shape=jax.ShapeDtypeStruct((batch, seq_len, d_model), x.dtype),
    )(x, encoding_weight)
